# baseline (speedup 1.0000x reference)
import jax
import jax.numpy as jnp
from jax import lax
from jax.experimental import pallas as pl
from jax.experimental.pallas import tpu as pltpu
from jax.experimental.pallas import tpu_sc as plsc


N = 10000
E = 320000
EH = E // 2
D = 64
DP = 128
SW = DP
NC, NS, L = 2, 16, 16
TILES = NC * NS
EPT = E // TILES
C = 40
NCHUNK = EPT // C
ISUP = 25
NSUP = NCHUNK // ISUP
NBUF = 3


def _node_body(nf, wn, bn, wa, x_out, xa_out):
    h = jnp.dot(nf[...], wn[...], preferred_element_type=jnp.float32) + bn[...]
    xx = jnp.maximum(h * jax.nn.sigmoid(h), 0.0)
    x_out[...] = xx
    xad = jnp.dot(xx, wa[...], preferred_element_type=jnp.float32)
    pad = (lax.broadcasted_iota(jnp.int32, (xx.shape[0], DP - D), 1) == 0)
    xa_out[...] = jnp.concatenate([xad, pad.astype(jnp.float32)], axis=1)


def _edge_body(ef0, ef1, we, be, wb, bm, ebb_out):
    def enc(ef):
        h = jnp.dot(ef, we[...], preferred_element_type=jnp.float32) + be[...]
        ee = jnp.maximum(h * jax.nn.sigmoid(h), 0.0)
        return jnp.dot(ee, wb[...], preferred_element_type=jnp.float32) + bm[...]
    ebb_out[...] = jnp.concatenate([enc(ef0[...]), enc(ef1[...])], axis=1)


def _final_body(s2, x, wm2, bm2, wua, wux, bu, wo, bo, out):
    stot = s2[0] + s2[1]
    aggr = (jnp.dot(stot[:, :D], wm2[...], preferred_element_type=jnp.float32)
            + stot[:, D:D + 1] * bm2[...])
    h = (jnp.dot(aggr, wua[...], preferred_element_type=jnp.float32)
         + jnp.dot(x[...], wux[...], preferred_element_type=jnp.float32)
         + bu[...])
    h = jnp.maximum(h * jax.nn.sigmoid(h), 0.0)
    out[...] = jnp.dot(h, wo[...], preferred_element_type=jnp.float32) + bo[...]


def _sc_edge_kernel(xa_hbm, gidx_hbm, ebb_hbm, zeros_hbm, out_hbm,
                    s_sh, gidx_v, ebb_v, gv, sems):
    cid = lax.axis_index("c")
    sid = lax.axis_index("s")
    tile = cid * NS + sid
    base = tile * EPT
    bbase = sid * EPT
    ecol = cid * D

    @pl.when(sid == 0)
    def _():
        pltpu.sync_copy(zeros_hbm, s_sh)

    plsc.subcore_barrier()

    def super_body(s, carry):
        srow = tile * NSUP + s
        pltpu.sync_copy(gidx_hbm.at[srow], gidx_v)

        def _loads(cix, b, issue):
            boffs = bbase + (s * ISUP + cix) * C
            mk = pltpu.async_copy if issue else pltpu.make_async_copy
            cps = [
                mk(ebb_hbm.at[pl.ds(boffs, C)], ebb_v.at[b], sems.at[b]),
                mk(xa_hbm.at[gidx_v.at[cix]], gv.at[b], sems.at[b]),
            ]
            if not issue:
                for cp in cps:
                    cp.wait()

        def _process(cix, b):
            _loads(cix, b, issue=False)

            def compute(ecol_static):
                def row_body(r, rc):
                    for j in range(D // L):
                        sl = pl.ds(j * L, L)
                        eb = ebb_v[b, r, pl.ds(ecol_static + j * L, L)]
                        u0 = gv[b, r, sl] + eb
                        u1 = gv[b, C + r, sl] + eb
                        gv[b, C + r, sl] = u0 / (1.0 + jnp.exp(-u0))
                        gv[b, r, sl] = u1 / (1.0 + jnp.exp(-u1))
                    return rc
                lax.fori_loop(0, C, row_body, 0)

            @pl.when(cid == 0)
            def _():
                compute(0)

            @pl.when(cid == 1)
            def _():
                compute(D)

            pltpu.sync_copy(gv.at[b], s_sh.at[gidx_v.at[cix]], add=True)

            more = cix + NBUF < ISUP
            if more is not False:
                @pl.when(more)
                def _():
                    _loads(cix + NBUF, b, issue=True)

        for b in range(NBUF):
            _loads(b, b, issue=True)

        def pair_body(p, carry2):
            for b in range(NBUF):
                _process(p * NBUF + b, b)
            return carry2

        lax.fori_loop(0, ISUP // NBUF, pair_body, 0)
        for t in range(ISUP - ISUP % NBUF, ISUP):
            _process(t, t % NBUF)
        return carry

    lax.fori_loop(0, NSUP, super_body, 0)

    plsc.subcore_barrier()

    @pl.when(sid == 0)
    def _():
        pltpu.sync_copy(s_sh, out_hbm.at[cid])


def kernel(node_features, edge_index, edge_features,
           W_n, b_n, W_e, b_e, Wm1, bm1, Wm2, bm2, Wu, bu, Wo, bo):
    nd = node_features.shape[1]
    ed = edge_features.shape[1]
    og = Wo.shape[1]

    wa = Wm1[:D]
    wb = Wm1[D:]
    bn2 = b_n.reshape(1, -1)
    be2 = b_e.reshape(1, -1)
    bm1r = bm1.reshape(1, -1)
    bm2r = bm2.reshape(1, -1)
    bu2 = bu.reshape(1, -1)
    wo_pad = jnp.zeros((D, 128), jnp.float32).at[:, :og].set(Wo)
    bo_pad = jnp.zeros((1, 128), jnp.float32).at[0, :og].set(bo)

    BN = 2000
    x, xa = pl.pallas_call(
        _node_body,
        grid=(N // BN,),
        in_specs=[pl.BlockSpec((BN, nd), lambda i: (i, 0)),
                  pl.BlockSpec((nd, D), lambda i: (0, 0)),
                  pl.BlockSpec((1, D), lambda i: (0, 0)),
                  pl.BlockSpec((D, D), lambda i: (0, 0))],
        out_specs=[pl.BlockSpec((BN, D), lambda i: (i, 0)),
                   pl.BlockSpec((BN, DP), lambda i: (i, 0))],
        out_shape=[jax.ShapeDtypeStruct((N, D), jnp.float32),
                   jax.ShapeDtypeStruct((N, DP), jnp.float32)],
    )(node_features, W_n, bn2, wa)

    BE = 8000
    nblk = (EH // BE)
    ebb = pl.pallas_call(
        _edge_body,
        grid=(nblk,),
        in_specs=[pl.BlockSpec((BE, ed), lambda i: (i, 0)),
                  pl.BlockSpec((BE, ed), lambda i: (i + nblk, 0)),
                  pl.BlockSpec((ed, ed), lambda i: (0, 0)),
                  pl.BlockSpec((1, ed), lambda i: (0, 0)),
                  pl.BlockSpec((ed, D), lambda i: (0, 0)),
                  pl.BlockSpec((1, D), lambda i: (0, 0))],
        out_specs=pl.BlockSpec((BE, 2 * D), lambda i: (i, 0)),
        out_shape=jax.ShapeDtypeStruct((EH, 2 * D), jnp.float32),
    )(edge_features, edge_features, W_e, be2, wb, bm1r)

    row3 = edge_index[0].reshape(-1, C)
    col3 = edge_index[1].reshape(-1, C)
    gidx = jnp.concatenate([row3, col3], axis=1).reshape(TILES * NSUP, ISUP, 2 * C)
    zeros = jnp.zeros((N, SW), jnp.float32)

    mesh = plsc.VectorSubcoreMesh(core_axis_name="c", subcore_axis_name="s",
                                  num_cores=NC, num_subcores=NS)
    s2 = pl.kernel(
        _sc_edge_kernel,
        out_type=jax.ShapeDtypeStruct((NC, N, SW), jnp.float32),
        mesh=mesh,
        scratch_types=[
            pltpu.VMEM_SHARED((N, SW), jnp.float32),
            pltpu.VMEM((ISUP, 2 * C), jnp.int32),
            pltpu.VMEM((NBUF, C, 2 * D), jnp.float32),
            pltpu.VMEM((NBUF, 2 * C, DP), jnp.float32),
            pltpu.SemaphoreType.DMA((NBUF,)),
        ],
    )(xa, gidx, ebb, zeros)

    out_pad = pl.pallas_call(
        _final_body,
        grid=(N // BN,),
        in_specs=[pl.BlockSpec((NC, BN, SW), lambda i: (0, i, 0)),
                  pl.BlockSpec((BN, D), lambda i: (i, 0)),
                  pl.BlockSpec((D, D), lambda i: (0, 0)),
                  pl.BlockSpec((1, D), lambda i: (0, 0)),
                  pl.BlockSpec((D, D), lambda i: (0, 0)),
                  pl.BlockSpec((D, D), lambda i: (0, 0)),
                  pl.BlockSpec((1, D), lambda i: (0, 0)),
                  pl.BlockSpec((D, 128), lambda i: (0, 0)),
                  pl.BlockSpec((1, 128), lambda i: (0, 0))],
        out_specs=pl.BlockSpec((BN, 128), lambda i: (i, 0)),
        out_shape=jax.ShapeDtypeStruct((N, 128), jnp.float32),
    )(s2, x, Wm2, bm2r, Wu[:D], Wu[D:], bu2, wo_pad, bo_pad)

    return out_pad[:, :og]


# speedup vs baseline: 1.0264x; 1.0264x over previous
import jax
import jax.numpy as jnp
from jax import lax
from jax.experimental import pallas as pl
from jax.experimental.pallas import tpu as pltpu
from jax.experimental.pallas import tpu_sc as plsc


N = 10000
E = 320000
EH = E // 2
D = 64
DP = 128
SW = DP
NC, NS, L = 2, 16, 16
TILES = NC * NS
EPT = E // TILES
C = 40
NCHUNK = EPT // C
ISUP = 50
NSUP = NCHUNK // ISUP
NBUF = 2


def _node_body(nf, wn, bn, wa, x_out, xa_out):
    h = jnp.dot(nf[...], wn[...], preferred_element_type=jnp.float32) + bn[...]
    xx = jnp.maximum(h * jax.nn.sigmoid(h), 0.0)
    x_out[...] = xx
    xad = jnp.dot(xx, wa[...], preferred_element_type=jnp.float32)
    pad = (lax.broadcasted_iota(jnp.int32, (xx.shape[0], DP - D), 1) == 0)
    xa_out[...] = jnp.concatenate([xad, pad.astype(jnp.float32)], axis=1)


def _edge_body(ef0, ef1, we, be, wb, bm, ebb_out):
    def enc(ef):
        h = jnp.dot(ef, we[...], preferred_element_type=jnp.float32) + be[...]
        ee = jnp.maximum(h * jax.nn.sigmoid(h), 0.0)
        return jnp.dot(ee, wb[...], preferred_element_type=jnp.float32) + bm[...]
    ebb_out[...] = jnp.concatenate([enc(ef0[...]), enc(ef1[...])], axis=1)


def _final_body(s2, x, wm2, bm2, wua, wux, bu, wo, bo, out):
    stot = s2[0] + s2[1]
    aggr = (jnp.dot(stot[:, :D], wm2[...], preferred_element_type=jnp.float32)
            + stot[:, D:D + 1] * bm2[...])
    h = (jnp.dot(aggr, wua[...], preferred_element_type=jnp.float32)
         + jnp.dot(x[...], wux[...], preferred_element_type=jnp.float32)
         + bu[...])
    h = jnp.maximum(h * jax.nn.sigmoid(h), 0.0)
    out[...] = jnp.dot(h, wo[...], preferred_element_type=jnp.float32) + bo[...]


def _sc_edge_kernel(xa_hbm, gidx_hbm, ebb_hbm, zeros_hbm, out_hbm,
                    s_sh, gidx_v, ebb_v, gv, sems):
    cid = lax.axis_index("c")
    sid = lax.axis_index("s")
    tile = cid * NS + sid
    base = tile * EPT
    bbase = sid * EPT
    ecol = cid * D

    @pl.when(sid == 0)
    def _():
        pltpu.sync_copy(zeros_hbm, s_sh)

    plsc.subcore_barrier()

    def super_body(s, carry):
        srow = tile * NSUP + s
        pltpu.sync_copy(gidx_hbm.at[srow], gidx_v)

        def _loads(cix, b, issue):
            boffs = bbase + (s * ISUP + cix) * C
            mk = pltpu.async_copy if issue else pltpu.make_async_copy
            cps = [
                mk(ebb_hbm.at[pl.ds(boffs, C)], ebb_v.at[b], sems.at[b]),
                mk(xa_hbm.at[gidx_v.at[cix]], gv.at[b], sems.at[b]),
            ]
            if not issue:
                for cp in cps:
                    cp.wait()

        def _process(cix, b):
            _loads(cix, b, issue=False)

            def compute(ecol_static):
                def row_body(r, rc):
                    for j in range(D // L):
                        sl = pl.ds(j * L, L)
                        eb = ebb_v[b, r, pl.ds(ecol_static + j * L, L)]
                        u0 = gv[b, r, sl] + eb
                        u1 = gv[b, C + r, sl] + eb
                        gv[b, C + r, sl] = u0 / (1.0 + jnp.exp(-u0))
                        gv[b, r, sl] = u1 / (1.0 + jnp.exp(-u1))
                    return rc
                lax.fori_loop(0, C, row_body, 0)

            @pl.when(cid == 0)
            def _():
                compute(0)

            @pl.when(cid == 1)
            def _():
                compute(D)

            pltpu.sync_copy(gv.at[b], s_sh.at[gidx_v.at[cix]], add=True)

            more = cix + NBUF < ISUP
            if more is not False:
                @pl.when(more)
                def _():
                    _loads(cix + NBUF, b, issue=True)

        for b in range(NBUF):
            _loads(b, b, issue=True)

        def pair_body(p, carry2):
            for b in range(NBUF):
                _process(p * NBUF + b, b)
            return carry2

        lax.fori_loop(0, ISUP // NBUF, pair_body, 0)
        for t in range(ISUP - ISUP % NBUF, ISUP):
            _process(t, t % NBUF)
        return carry

    lax.fori_loop(0, NSUP, super_body, 0)

    plsc.subcore_barrier()

    @pl.when(sid == 0)
    def _():
        pltpu.sync_copy(s_sh, out_hbm.at[cid])


def kernel(node_features, edge_index, edge_features,
           W_n, b_n, W_e, b_e, Wm1, bm1, Wm2, bm2, Wu, bu, Wo, bo):
    nd = node_features.shape[1]
    ed = edge_features.shape[1]
    og = Wo.shape[1]

    wa = Wm1[:D]
    wb = Wm1[D:]
    bn2 = b_n.reshape(1, -1)
    be2 = b_e.reshape(1, -1)
    bm1r = bm1.reshape(1, -1)
    bm2r = bm2.reshape(1, -1)
    bu2 = bu.reshape(1, -1)
    wo_pad = jnp.zeros((D, 128), jnp.float32).at[:, :og].set(Wo)
    bo_pad = jnp.zeros((1, 128), jnp.float32).at[0, :og].set(bo)

    BN = 2000
    x, xa = pl.pallas_call(
        _node_body,
        grid=(N // BN,),
        in_specs=[pl.BlockSpec((BN, nd), lambda i: (i, 0)),
                  pl.BlockSpec((nd, D), lambda i: (0, 0)),
                  pl.BlockSpec((1, D), lambda i: (0, 0)),
                  pl.BlockSpec((D, D), lambda i: (0, 0))],
        out_specs=[pl.BlockSpec((BN, D), lambda i: (i, 0)),
                   pl.BlockSpec((BN, DP), lambda i: (i, 0))],
        out_shape=[jax.ShapeDtypeStruct((N, D), jnp.float32),
                   jax.ShapeDtypeStruct((N, DP), jnp.float32)],
    )(node_features, W_n, bn2, wa)

    BE = 8000
    nblk = (EH // BE)
    ebb = pl.pallas_call(
        _edge_body,
        grid=(nblk,),
        in_specs=[pl.BlockSpec((BE, ed), lambda i: (i, 0)),
                  pl.BlockSpec((BE, ed), lambda i: (i + nblk, 0)),
                  pl.BlockSpec((ed, ed), lambda i: (0, 0)),
                  pl.BlockSpec((1, ed), lambda i: (0, 0)),
                  pl.BlockSpec((ed, D), lambda i: (0, 0)),
                  pl.BlockSpec((1, D), lambda i: (0, 0))],
        out_specs=pl.BlockSpec((BE, 2 * D), lambda i: (i, 0)),
        out_shape=jax.ShapeDtypeStruct((EH, 2 * D), jnp.float32),
    )(edge_features, edge_features, W_e, be2, wb, bm1r)

    row3 = edge_index[0].reshape(-1, C)
    col3 = edge_index[1].reshape(-1, C)
    gidx = jnp.concatenate([row3, col3], axis=1).reshape(TILES * NSUP, ISUP, 2 * C)
    zeros = jnp.zeros((N, SW), jnp.float32)

    mesh = plsc.VectorSubcoreMesh(core_axis_name="c", subcore_axis_name="s",
                                  num_cores=NC, num_subcores=NS)
    s2 = pl.kernel(
        _sc_edge_kernel,
        out_type=jax.ShapeDtypeStruct((NC, N, SW), jnp.float32),
        mesh=mesh,
        scratch_types=[
            pltpu.VMEM_SHARED((N, SW), jnp.float32),
            pltpu.VMEM((ISUP, 2 * C), jnp.int32),
            pltpu.VMEM((NBUF, C, 2 * D), jnp.float32),
            pltpu.VMEM((NBUF, 2 * C, DP), jnp.float32),
            pltpu.SemaphoreType.DMA((NBUF,)),
        ],
    )(xa, gidx, ebb, zeros)

    out_pad = pl.pallas_call(
        _final_body,
        grid=(N // BN,),
        in_specs=[pl.BlockSpec((NC, BN, SW), lambda i: (0, i, 0)),
                  pl.BlockSpec((BN, D), lambda i: (i, 0)),
                  pl.BlockSpec((D, D), lambda i: (0, 0)),
                  pl.BlockSpec((1, D), lambda i: (0, 0)),
                  pl.BlockSpec((D, D), lambda i: (0, 0)),
                  pl.BlockSpec((D, D), lambda i: (0, 0)),
                  pl.BlockSpec((1, D), lambda i: (0, 0)),
                  pl.BlockSpec((D, 128), lambda i: (0, 0)),
                  pl.BlockSpec((1, 128), lambda i: (0, 0))],
        out_specs=pl.BlockSpec((BN, 128), lambda i: (i, 0)),
        out_shape=jax.ShapeDtypeStruct((N, 128), jnp.float32),
    )(s2, x, Wm2, bm2r, Wu[:D], Wu[D:], bu2, wo_pad, bo_pad)

    return out_pad[:, :og]


# speedup vs baseline: 1.1086x; 1.0801x over previous
import jax
import jax.numpy as jnp
from jax import lax
from jax.experimental import pallas as pl
from jax.experimental.pallas import tpu as pltpu
from jax.experimental.pallas import tpu_sc as plsc


N = 10000
E = 320000
EH = E // 2
D = 64
DP = 128
SW = DP
NC, NS, L = 2, 16, 16
TILES = NC * NS
EPT = E // TILES
C = 40
NCHUNK = EPT // C
ISUP = 50
NSUP = NCHUNK // ISUP
NBUF = 2


def _node_body(nf, wn, bn, wa, x_out, xa_out):
    h = jnp.dot(nf[...], wn[...], preferred_element_type=jnp.float32) + bn[...]
    xx = jnp.maximum(h * jax.nn.sigmoid(h), 0.0)
    x_out[...] = xx
    xad = jnp.dot(xx, wa[...], preferred_element_type=jnp.float32)
    pad = (lax.broadcasted_iota(jnp.int32, (xx.shape[0], DP - D), 1) == 0)
    xa_out[...] = jnp.concatenate([xad, pad.astype(jnp.float32)], axis=1)


def _edge_body(ef0, ef1, we, be, wb, bm, ebb_out):
    def enc(ef):
        h = jnp.dot(ef, we[...], preferred_element_type=jnp.float32) + be[...]
        ee = jnp.maximum(h * jax.nn.sigmoid(h), 0.0)
        return jnp.dot(ee, wb[...], preferred_element_type=jnp.float32) + bm[...]
    ebb_out[...] = jnp.concatenate([enc(ef0[...]), enc(ef1[...])], axis=1)


def _final_body(s2, x, wm2, bm2, wua, wux, bu, wo, bo, out):
    stot = s2[0] + s2[1]
    aggr = (jnp.dot(stot[:, :D], wm2[...], preferred_element_type=jnp.float32)
            + stot[:, D:D + 1] * bm2[...])
    h = (jnp.dot(aggr, wua[...], preferred_element_type=jnp.float32)
         + jnp.dot(x[...], wux[...], preferred_element_type=jnp.float32)
         + bu[...])
    h = jnp.maximum(h * jax.nn.sigmoid(h), 0.0)
    out[...] = jnp.dot(h, wo[...], preferred_element_type=jnp.float32) + bo[...]


def _sc_edge_kernel(xa_hbm, gidx_hbm, ebb_hbm, zeros_hbm, out_hbm,
                    s_sh, gidx_v, ebb_v, gv, sems):
    cid = lax.axis_index("c")
    sid = lax.axis_index("s")
    tile = cid * NS + sid
    base = tile * EPT
    bbase = sid * EPT
    ecol = cid * D

    @pl.when(sid == 0)
    def _():
        pltpu.sync_copy(zeros_hbm, s_sh)

    plsc.subcore_barrier()

    def super_body(s, carry):
        srow = tile * NSUP + s
        pltpu.sync_copy(gidx_hbm.at[srow], gidx_v)

        def _loads(cix, b, issue):
            boffs = bbase + (s * ISUP + cix) * C
            mk = pltpu.async_copy if issue else pltpu.make_async_copy
            cps = [
                mk(ebb_hbm.at[pl.ds(boffs, C)], ebb_v.at[b], sems.at[b]),
                mk(xa_hbm.at[gidx_v.at[cix]], gv.at[b], sems.at[b]),
            ]
            if not issue:
                for cp in cps:
                    cp.wait()

        def _process(cix, b):
            _loads(cix, b, issue=False)

            def compute(ecol_static):
                @plsc.parallel_loop(0, C, 1, unroll=2)
                def row_body(r):
                    for j in range(D // L):
                        sl = pl.ds(j * L, L)
                        eb = ebb_v[b, r, pl.ds(ecol_static + j * L, L)]
                        u0 = gv[b, r, sl] + eb
                        u1 = gv[b, C + r, sl] + eb
                        gv[b, C + r, sl] = u0 / (1.0 + jnp.exp(-u0))
                        gv[b, r, sl] = u1 / (1.0 + jnp.exp(-u1))

            @pl.when(cid == 0)
            def _():
                compute(0)

            @pl.when(cid == 1)
            def _():
                compute(D)

            pltpu.sync_copy(gv.at[b], s_sh.at[gidx_v.at[cix]], add=True)

            more = cix + NBUF < ISUP
            if more is not False:
                @pl.when(more)
                def _():
                    _loads(cix + NBUF, b, issue=True)

        for b in range(NBUF):
            _loads(b, b, issue=True)

        def pair_body(p, carry2):
            for b in range(NBUF):
                _process(p * NBUF + b, b)
            return carry2

        lax.fori_loop(0, ISUP // NBUF, pair_body, 0)
        for t in range(ISUP - ISUP % NBUF, ISUP):
            _process(t, t % NBUF)
        return carry

    lax.fori_loop(0, NSUP, super_body, 0)

    plsc.subcore_barrier()

    @pl.when(sid == 0)
    def _():
        pltpu.sync_copy(s_sh, out_hbm.at[cid])


def kernel(node_features, edge_index, edge_features,
           W_n, b_n, W_e, b_e, Wm1, bm1, Wm2, bm2, Wu, bu, Wo, bo):
    nd = node_features.shape[1]
    ed = edge_features.shape[1]
    og = Wo.shape[1]

    wa = Wm1[:D]
    wb = Wm1[D:]
    bn2 = b_n.reshape(1, -1)
    be2 = b_e.reshape(1, -1)
    bm1r = bm1.reshape(1, -1)
    bm2r = bm2.reshape(1, -1)
    bu2 = bu.reshape(1, -1)
    wo_pad = jnp.zeros((D, 128), jnp.float32).at[:, :og].set(Wo)
    bo_pad = jnp.zeros((1, 128), jnp.float32).at[0, :og].set(bo)

    BN = 2000
    x, xa = pl.pallas_call(
        _node_body,
        grid=(N // BN,),
        in_specs=[pl.BlockSpec((BN, nd), lambda i: (i, 0)),
                  pl.BlockSpec((nd, D), lambda i: (0, 0)),
                  pl.BlockSpec((1, D), lambda i: (0, 0)),
                  pl.BlockSpec((D, D), lambda i: (0, 0))],
        out_specs=[pl.BlockSpec((BN, D), lambda i: (i, 0)),
                   pl.BlockSpec((BN, DP), lambda i: (i, 0))],
        out_shape=[jax.ShapeDtypeStruct((N, D), jnp.float32),
                   jax.ShapeDtypeStruct((N, DP), jnp.float32)],
    )(node_features, W_n, bn2, wa)

    BE = 8000
    nblk = (EH // BE)
    ebb = pl.pallas_call(
        _edge_body,
        grid=(nblk,),
        in_specs=[pl.BlockSpec((BE, ed), lambda i: (i, 0)),
                  pl.BlockSpec((BE, ed), lambda i: (i + nblk, 0)),
                  pl.BlockSpec((ed, ed), lambda i: (0, 0)),
                  pl.BlockSpec((1, ed), lambda i: (0, 0)),
                  pl.BlockSpec((ed, D), lambda i: (0, 0)),
                  pl.BlockSpec((1, D), lambda i: (0, 0))],
        out_specs=pl.BlockSpec((BE, 2 * D), lambda i: (i, 0)),
        out_shape=jax.ShapeDtypeStruct((EH, 2 * D), jnp.float32),
    )(edge_features, edge_features, W_e, be2, wb, bm1r)

    row3 = edge_index[0].reshape(-1, C)
    col3 = edge_index[1].reshape(-1, C)
    gidx = jnp.concatenate([row3, col3], axis=1).reshape(TILES * NSUP, ISUP, 2 * C)
    zeros = jnp.zeros((N, SW), jnp.float32)

    mesh = plsc.VectorSubcoreMesh(core_axis_name="c", subcore_axis_name="s",
                                  num_cores=NC, num_subcores=NS)
    s2 = pl.kernel(
        _sc_edge_kernel,
        out_type=jax.ShapeDtypeStruct((NC, N, SW), jnp.float32),
        mesh=mesh,
        scratch_types=[
            pltpu.VMEM_SHARED((N, SW), jnp.float32),
            pltpu.VMEM((ISUP, 2 * C), jnp.int32),
            pltpu.VMEM((NBUF, C, 2 * D), jnp.float32),
            pltpu.VMEM((NBUF, 2 * C, DP), jnp.float32),
            pltpu.SemaphoreType.DMA((NBUF,)),
        ],
    )(xa, gidx, ebb, zeros)

    out_pad = pl.pallas_call(
        _final_body,
        grid=(N // BN,),
        in_specs=[pl.BlockSpec((NC, BN, SW), lambda i: (0, i, 0)),
                  pl.BlockSpec((BN, D), lambda i: (i, 0)),
                  pl.BlockSpec((D, D), lambda i: (0, 0)),
                  pl.BlockSpec((1, D), lambda i: (0, 0)),
                  pl.BlockSpec((D, D), lambda i: (0, 0)),
                  pl.BlockSpec((D, D), lambda i: (0, 0)),
                  pl.BlockSpec((1, D), lambda i: (0, 0)),
                  pl.BlockSpec((D, 128), lambda i: (0, 0)),
                  pl.BlockSpec((1, 128), lambda i: (0, 0))],
        out_specs=pl.BlockSpec((BN, 128), lambda i: (i, 0)),
        out_shape=jax.ShapeDtypeStruct((N, 128), jnp.float32),
    )(s2, x, Wm2, bm2r, Wu[:D], Wu[D:], bu2, wo_pad, bo_pad)

    return out_pad[:, :og]
